# baseline (speedup 1.0000x reference)
import functools

import jax
import jax.numpy as jnp
from jax import lax
from jax.experimental import pallas as pl
from jax.experimental.pallas import tpu as pltpu
from jax.experimental.pallas import tpu_sc as plsc

NC, NS = 2, 16
NW = NC * NS
LANES = 16
CHUNK = 128
NBUF = 5
GAT_AHEAD = 3


@functools.lru_cache(maxsize=None)
def _build(batch, seq_len, dim):
    rows = batch * seq_len
    span = seq_len // NW
    nch = (batch * span) // CHUNK
    ch_per_b = span // CHUNK
    nvec = dim // LANES
    mesh = plsc.VectorSubcoreMesh(
        core_axis_name="c", subcore_axis_name="s",
        num_cores=NC, num_subcores=NS)

    @functools.partial(
        pl.kernel,
        out_type=jax.ShapeDtypeStruct((rows, dim), jnp.float32),
        mesh=mesh,
        scratch_types=[
            pltpu.VMEM((batch, span), jnp.int32),
            pltpu.VMEM((span, dim), jnp.float32),
            pltpu.VMEM((NBUF, CHUNK, dim), jnp.float32),
            pltpu.SemaphoreType.DMA,
            pltpu.SemaphoreType.DMA,
            pltpu.SemaphoreType.DMA((2,)),
            pltpu.SemaphoreType.DMA((NBUF,)),
            pltpu.SemaphoreType.DMA((NBUF,)),
        ],
    )
    def emb(ids_hbm, table_hbm, pos_hbm, out_hbm, idx_v, pos_v, bufs,
            hd_sem, id_sem, ld_sem, gat_sem, out_sem):
        wid = lax.axis_index("s") * NC + lax.axis_index("c")
        s0 = wid * span

        idx_head = pltpu.async_copy(
            ids_hbm.at[0, pl.ds(s0, 32)], idx_v.at[0, pl.ds(0, 32)],
            hd_sem)
        idx_rest = pltpu.async_copy(
            ids_hbm.at[:, pl.ds(s0, span)], idx_v, id_sem)
        pos_lds = [
            pltpu.async_copy(pos_hbm.at[pl.ds(s0 + h * CHUNK, CHUNK)],
                             pos_v.at[pl.ds(h * CHUNK, CHUNK)], ld_sem.at[h])
            for h in range(ch_per_b)]

        pend_gat, pend_out = {}, {}

        chunks = [(0, 0, 32), (0, 32, 96)]
        chunks += [(b_row, 0, CHUNK) for b_row in range(1, batch)]
        chunks += [(b_row, CHUNK, CHUNK) for b_row in range(batch - 1)]
        chunks += [(batch - 1, CHUNK, 96), (batch - 1, CHUNK + 96, 32)]
        ncs = len(chunks)

        def start_gather(c):
            b = c % NBUF
            if c - NBUF in pend_out:
                pend_out.pop(c - NBUF).wait()
            b_row, roff, n = chunks[c]
            pend_gat[c] = pltpu.async_copy(
                table_hbm.at[idx_v.at[b_row, pl.ds(roff, n)]],
                bufs.at[b, pl.ds(0, n)], gat_sem.at[b])

        def add_pos(c):
            b = c % NBUF
            _, roff, n = chunks[c]
            buf = bufs.at[b]

            @plsc.parallel_loop(0, n, unroll=2)
            def body(r):
                pr = roff + r
                for j in range(nvec):
                    sl = pl.ds(j * LANES, LANES)
                    plsc.addupdate(buf.at[r, sl], pos_v[pr, sl])

        def finish_chunk(c):
            b = c % NBUF
            b_row, roff, n = chunks[c]
            half = roff // CHUNK
            if pos_lds[half] is not None:
                pos_lds[half].wait()
                pos_lds[half] = None
            pend_gat.pop(c).wait()
            add_pos(c)
            pend_out[c] = pltpu.async_copy(
                bufs.at[b, pl.ds(0, n)],
                out_hbm.at[pl.ds(b_row * seq_len + s0 + roff, n)],
                out_sem.at[b])

        idx_head.wait()
        start_gather(0)
        idx_rest.wait()
        for c in range(1, GAT_AHEAD):
            start_gather(c)
        for c in range(ncs):
            if c + GAT_AHEAD < ncs:
                start_gather(c + GAT_AHEAD)
            finish_chunk(c)
        for c in sorted(pend_out):
            pend_out.pop(c).wait()

    return emb


def kernel(input_ids, embedding_weight, pos_embedding):
    batch, seq_len = input_ids.shape
    _, dim = embedding_weight.shape
    ids = input_ids.astype(jnp.int32)
    out = _build(batch, seq_len, dim)(ids, embedding_weight, pos_embedding)
    return out.reshape(batch, seq_len, dim)


# speedup vs baseline: 1.0253x; 1.0253x over previous
import functools

import jax
import jax.numpy as jnp
from jax import lax
from jax.experimental import pallas as pl
from jax.experimental.pallas import tpu as pltpu
from jax.experimental.pallas import tpu_sc as plsc

NC, NS = 2, 16
NW = NC * NS
LANES = 16
CHUNK = 128
NBUF = 5
GAT_AHEAD = 3


@functools.lru_cache(maxsize=None)
def _build(batch, seq_len, dim):
    rows = batch * seq_len
    span = seq_len // NW
    nch = (batch * span) // CHUNK
    ch_per_b = span // CHUNK
    nvec = dim // LANES
    mesh = plsc.VectorSubcoreMesh(
        core_axis_name="c", subcore_axis_name="s",
        num_cores=NC, num_subcores=NS)

    @functools.partial(
        pl.kernel,
        out_type=jax.ShapeDtypeStruct((rows, dim), jnp.float32),
        mesh=mesh,
        scratch_types=[
            pltpu.VMEM((batch, span), jnp.int32),
            pltpu.VMEM((span, dim), jnp.float32),
            pltpu.VMEM((NBUF, CHUNK, dim), jnp.float32),
            pltpu.SemaphoreType.DMA,
            pltpu.SemaphoreType.DMA,
            pltpu.SemaphoreType.DMA((2,)),
            pltpu.SemaphoreType.DMA((NBUF,)),
            pltpu.SemaphoreType.DMA((NBUF,)),
        ],
    )
    def emb(ids_hbm, table_hbm, pos_hbm, out_hbm, idx_v, pos_v, bufs,
            hd_sem, id_sem, ld_sem, gat_sem, out_sem):
        wid = lax.axis_index("s") * NC + lax.axis_index("c")
        s0 = wid * span

        idx_head = pltpu.async_copy(
            ids_hbm.at[0, pl.ds(s0, 32)], idx_v.at[0, pl.ds(0, 32)],
            hd_sem)
        idx_rest = pltpu.async_copy(
            ids_hbm.at[:, pl.ds(s0, span)], idx_v, id_sem)
        pos_lds = [
            pltpu.async_copy(pos_hbm.at[pl.ds(s0 + h * CHUNK, CHUNK)],
                             pos_v.at[pl.ds(h * CHUNK, CHUNK)], ld_sem.at[h])
            for h in range(ch_per_b)]

        pend_gat, pend_out = {}, {}

        chunks = [(0, 0, 32), (0, 32, 96)]
        chunks += [(b_row, 0, CHUNK) for b_row in range(1, batch)]
        chunks += [(b_row, CHUNK, CHUNK) for b_row in range(batch - 1)]
        chunks += [(batch - 1, CHUNK, 96), (batch - 1, CHUNK + 96, 32)]
        ncs = len(chunks)

        def start_gather(c):
            b = c % NBUF
            if c - NBUF in pend_out:
                pend_out.pop(c - NBUF).wait()
            b_row, roff, n = chunks[c]
            pend_gat[c] = pltpu.async_copy(
                table_hbm.at[idx_v.at[b_row, pl.ds(roff, n)]],
                bufs.at[b, pl.ds(0, n)], gat_sem.at[b])

        def add_pos(c):
            b = c % NBUF
            _, roff, n = chunks[c]
            buf = bufs.at[b]

            def body(r, carry):
                pr = roff + r
                for j in range(nvec):
                    sl = pl.ds(j * LANES, LANES)
                    plsc.addupdate(buf.at[r, sl], pos_v[pr, sl])
                return carry

            lax.fori_loop(0, n, body, 0)

        def finish_chunk(c):
            b = c % NBUF
            b_row, roff, n = chunks[c]
            half = roff // CHUNK
            if pos_lds[half] is not None:
                pos_lds[half].wait()
                pos_lds[half] = None
            pend_gat.pop(c).wait()
            add_pos(c)
            pend_out[c] = pltpu.async_copy(
                bufs.at[b, pl.ds(0, n)],
                out_hbm.at[pl.ds(b_row * seq_len + s0 + roff, n)],
                out_sem.at[b])

        idx_head.wait()
        start_gather(0)
        idx_rest.wait()
        for c in range(1, GAT_AHEAD):
            start_gather(c)
        for c in range(ncs):
            if c + GAT_AHEAD < ncs:
                start_gather(c + GAT_AHEAD)
            finish_chunk(c)
        for c in sorted(pend_out):
            pend_out.pop(c).wait()

    return emb


def kernel(input_ids, embedding_weight, pos_embedding):
    batch, seq_len = input_ids.shape
    _, dim = embedding_weight.shape
    ids = input_ids.astype(jnp.int32)
    out = _build(batch, seq_len, dim)(ids, embedding_weight, pos_embedding)
    return out.reshape(batch, seq_len, dim)
